# Initial kernel scaffold; baseline (speedup 1.0000x reference)
import jax
import jax.numpy as jnp
from jax.experimental import pallas as pl


def kernel(boxes, scores):
    raise NotImplementedError("write your pallas kernel here")


# speedup vs baseline: 1.8999x; 1.8999x over previous
import functools

import jax
import jax.numpy as jnp
from jax.experimental import pallas as pl

_R = 8
_IOU_TH = 0.5
_SCORE_TH = 0.5


def _nms_kernel(s_ref, c_ref, os_ref, oy1_ref, ox1_ref, oy2_ref, ox2_ref,
                *, C, NP):
    s = jnp.reshape(s_ref[...], (C, NP))
    s = jnp.where(s > _SCORE_TH, s, -1.0)
    c = jnp.reshape(c_ref[...], (4, NP))
    b0, b1, b2, b3 = c[0:1, :], c[1:2, :], c[2:3, :], c[3:4, :]
    y1 = jnp.minimum(b0, b2)
    x1 = jnp.minimum(b1, b3)
    y2 = jnp.maximum(b0, b2)
    x2 = jnp.maximum(b1, b3)
    area = (y2 - y1) * (x2 - x1)
    iota = jax.lax.broadcasted_iota(jnp.int32, (C, NP), 1)
    sS, sY1, sX1, sY2, sX2 = [], [], [], [], []
    for _ in range(_R):
        m = jnp.max(s, axis=1, keepdims=True)
        idx = jnp.min(jnp.where(s == m, iota, NP), axis=1, keepdims=True)
        onehot = iota == idx
        gy1 = jnp.sum(jnp.where(onehot, y1, 0.0), axis=1, keepdims=True)
        gx1 = jnp.sum(jnp.where(onehot, x1, 0.0), axis=1, keepdims=True)
        gy2 = jnp.sum(jnp.where(onehot, y2, 0.0), axis=1, keepdims=True)
        gx2 = jnp.sum(jnp.where(onehot, x2, 0.0), axis=1, keepdims=True)
        valid = m > 0.0
        sS.append(jnp.where(valid, m, 0.0))
        sY1.append(jnp.where(valid, gy1, 0.0))
        sX1.append(jnp.where(valid, gx1, 0.0))
        sY2.append(jnp.where(valid, gy2, 0.0))
        sX2.append(jnp.where(valid, gx2, 0.0))
        iy1 = jnp.maximum(gy1, y1)
        ix1 = jnp.maximum(gx1, x1)
        iy2 = jnp.minimum(gy2, y2)
        ix2 = jnp.minimum(gx2, x2)
        inter = jnp.maximum(iy2 - iy1, 0.0) * jnp.maximum(ix2 - ix1, 0.0)
        sarea = (gy2 - gy1) * (gx2 - gx1)
        union = sarea + area - inter
        s = jnp.where((inter > _IOU_TH * union) & valid, -1.0, s)
    os_ref[...] = jnp.reshape(jnp.concatenate(sS, axis=1), (1, C, _R))
    oy1_ref[...] = jnp.reshape(jnp.concatenate(sY1, axis=1), (1, C, _R))
    ox1_ref[...] = jnp.reshape(jnp.concatenate(sX1, axis=1), (1, C, _R))
    oy2_ref[...] = jnp.reshape(jnp.concatenate(sY2, axis=1), (1, C, _R))
    ox2_ref[...] = jnp.reshape(jnp.concatenate(sX2, axis=1), (1, C, _R))


def _merge_kernel(s_ref, y1_ref, x1_ref, y2_ref, x2_ref,
                  os_ref, oy1_ref, ox1_ref, oy2_ref, ox2_ref,
                  ocls_ref, ovd_ref):
    B, M = s_ref.shape
    s = s_ref[...]
    y1, x1, y2, x2 = y1_ref[...], x1_ref[...], y2_ref[...], x2_ref[...]
    iota = jax.lax.broadcasted_iota(jnp.int32, (B, M), 1)
    vd = jnp.zeros((B, 1), jnp.int32)
    oS, oY1, oX1, oY2, oX2, oC = [], [], [], [], [], []
    for _ in range(_R):
        m = jnp.max(s, axis=1, keepdims=True)
        idx = jnp.min(jnp.where(s == m, iota, M), axis=1, keepdims=True)
        onehot = iota == idx
        gy1 = jnp.sum(jnp.where(onehot, y1, 0.0), axis=1, keepdims=True)
        gx1 = jnp.sum(jnp.where(onehot, x1, 0.0), axis=1, keepdims=True)
        gy2 = jnp.sum(jnp.where(onehot, y2, 0.0), axis=1, keepdims=True)
        gx2 = jnp.sum(jnp.where(onehot, x2, 0.0), axis=1, keepdims=True)
        valid = m > 0.0
        oS.append(jnp.where(valid, m, 0.0))
        oY1.append(jnp.clip(jnp.where(valid, gy1, 0.0), 0.0, 1.0))
        oX1.append(jnp.clip(jnp.where(valid, gx1, 0.0), 0.0, 1.0))
        oY2.append(jnp.clip(jnp.where(valid, gy2, 0.0), 0.0, 1.0))
        oX2.append(jnp.clip(jnp.where(valid, gx2, 0.0), 0.0, 1.0))
        oC.append(jnp.where(valid, idx // _R, 0))
        vd = vd + valid.astype(jnp.int32)
        s = jnp.where(onehot, -1.0, s)
    os_ref[...] = jnp.concatenate(oS, axis=1)
    oy1_ref[...] = jnp.concatenate(oY1, axis=1)
    ox1_ref[...] = jnp.concatenate(oX1, axis=1)
    oy2_ref[...] = jnp.concatenate(oY2, axis=1)
    ox2_ref[...] = jnp.concatenate(oX2, axis=1)
    ocls_ref[...] = jnp.concatenate(oC, axis=1)
    ovd_ref[...] = vd


@jax.jit
def kernel(boxes, scores):
    B, N, _, _ = boxes.shape
    C = scores.shape[-1]
    NP = ((N + 127) // 128) * 128
    f32 = jnp.float32
    s_t = jnp.transpose(scores, (0, 2, 1))
    s_t = jnp.pad(s_t, ((0, 0), (0, 0), (0, NP - N)))
    c_t = jnp.transpose(boxes[:, :, 0, :], (0, 2, 1))
    c_t = jnp.pad(c_t, ((0, 0), (0, 0), (0, NP - N)))

    nms = pl.pallas_call(
        functools.partial(_nms_kernel, C=C, NP=NP),
        grid=(B,),
        in_specs=[
            pl.BlockSpec((1, C, NP), lambda b: (b, 0, 0)),
            pl.BlockSpec((1, 4, NP), lambda b: (b, 0, 0)),
        ],
        out_specs=[pl.BlockSpec((1, C, _R), lambda b: (b, 0, 0))] * 5,
        out_shape=[jax.ShapeDtypeStruct((B, C, _R), f32)] * 5,
    )
    cs, cy1, cx1, cy2, cx2 = nms(s_t, c_t)

    M = C * _R
    merge = pl.pallas_call(
        _merge_kernel,
        out_shape=(
            [jax.ShapeDtypeStruct((B, _R), f32)] * 5
            + [jax.ShapeDtypeStruct((B, _R), jnp.int32),
               jax.ShapeDtypeStruct((B, 1), jnp.int32)]
        ),
    )
    ms, my1, mx1, my2, mx2, mcls, mvd = merge(
        cs.reshape(B, M), cy1.reshape(B, M), cx1.reshape(B, M),
        cy2.reshape(B, M), cx2.reshape(B, M))
    out_boxes = jnp.stack([my1, mx1, my2, mx2], axis=-1)
    return out_boxes, ms, mcls, mvd[:, 0]
